# Initial kernel scaffold; baseline (speedup 1.0000x reference)
import jax
import jax.numpy as jnp
from jax.experimental import pallas as pl


def kernel(x, V, E, W_in, b_in, W0, alpha0, beta0, g0, bln0, W1, alpha1, beta1, g1, bln1, W_out, b_out):
    raise NotImplementedError("write your pallas kernel here")


# speedup vs baseline: 4.7017x; 4.7017x over previous
import functools

import jax
import jax.numpy as jnp
from jax import lax
from jax.experimental import pallas as pl
from jax.experimental.pallas import tpu as pltpu, tpu_sc as plsc

N = 10000
NNZ = 320000
NE = 5000
F = 128
NCLASS = 16
VP = 10240
EP = 5120
DEGE = 0.5
EPS = 1e-5

NTILES = 32
PAIRS_PER_TILE = NNZ // NTILES
C = 80
NCHUNK = PAIRS_PER_TILE // C

_mesh = plsc.VectorSubcoreMesh(core_axis_name="c", subcore_axis_name="s")


def _make_seg_kernel(ndst, with_counts):
    rows_per_tile = ndst // 16
    crows = EP // 16
    out_type = [jax.ShapeDtypeStruct((2, ndst, F), jnp.float32)]
    scratch = [
        pltpu.VMEM_SHARED((ndst, F), jnp.float32),
        pltpu.VMEM((C,), jnp.int32),
        pltpu.VMEM((C,), jnp.int32),
        pltpu.VMEM((C, F), jnp.float32),
        pltpu.SemaphoreType.DMA,
    ]
    if with_counts:
        out_type.append(jax.ShapeDtypeStruct((2, EP, 16), jnp.float32))
        scratch += [
            pltpu.VMEM_SHARED((EP, 16), jnp.float32),
            pltpu.VMEM((C, 16), jnp.float32),
        ]

    def body(table, src_idx, dst_idx, zeros, *rest):
        if with_counts:
            ones, zeros16, out, cnt_out, accum, sidx, didx, rows, sem, cnt_sh, ones_v = rest
        else:
            out, accum, sidx, didx, rows, sem = rest
        core = lax.axis_index("c")
        sub = lax.axis_index("s")
        wid = core * 16 + sub
        r0 = sub * rows_per_tile
        pltpu.sync_copy(zeros.at[pl.ds(r0, rows_per_tile)],
                        accum.at[pl.ds(r0, rows_per_tile)])
        if with_counts:
            pltpu.sync_copy(zeros16.at[pl.ds(sub * crows, crows)],
                            cnt_sh.at[pl.ds(sub * crows, crows)])
            pltpu.sync_copy(ones, ones_v)
        plsc.subcore_barrier()

        def step(k, carry):
            base = wid * PAIRS_PER_TILE + k * C
            pltpu.sync_copy(src_idx.at[pl.ds(base, C)], sidx)
            pltpu.sync_copy(dst_idx.at[pl.ds(base, C)], didx)
            pltpu.async_copy(table.at[sidx], rows, sem).wait()
            pltpu.sync_copy(rows, accum.at[didx], add=True)
            if with_counts:
                pltpu.sync_copy(ones_v, cnt_sh.at[didx], add=True)
            return carry

        lax.fori_loop(0, NCHUNK, step, 0)
        plsc.subcore_barrier()
        pltpu.sync_copy(accum.at[pl.ds(r0, rows_per_tile)],
                        out.at[core, pl.ds(r0, rows_per_tile)])
        if with_counts:
            pltpu.sync_copy(cnt_sh.at[pl.ds(sub * crows, crows)],
                            cnt_out.at[core, pl.ds(sub * crows, crows)])

    return pl.kernel(body, mesh=_mesh, out_type=out_type, scratch_types=scratch)


_seg_edge_cnt = _make_seg_kernel(EP, True)
_seg_edge = _make_seg_kernel(EP, False)
_seg_vert = _make_seg_kernel(VP, False)


BR = 2048
BRE = 1024


def _lin_silu_body(x_ref, w_ref, b_ref, o_ref):
    z = lax.dot_general(x_ref[...], w_ref[...], (((1,), (1,)), ((), ())),
                        preferred_element_type=jnp.float32)
    z = z + b_ref[...]
    o_ref[...] = z * jax.nn.sigmoid(z)


def _lin_silu(x, w, b):
    return pl.pallas_call(
        _lin_silu_body,
        grid=(VP // BR,),
        in_specs=[
            pl.BlockSpec((BR, F), lambda i: (i, 0)),
            pl.BlockSpec((F, F), lambda i: (0, 0)),
            pl.BlockSpec((1, F), lambda i: (0, 0)),
        ],
        out_specs=pl.BlockSpec((BR, F), lambda i: (i, 0)),
        out_shape=jax.ShapeDtypeStruct((VP, F), jnp.float32),
    )(x, w, b)


def _edge_combine_body(p_ref, c_ref, o_ref):
    s = p_ref[0] + p_ref[1]
    cnt = c_ref[0, :, 0:1] + c_ref[1, :, 0:1]
    o_ref[...] = s * (DEGE / jnp.maximum(cnt, 1.0))


def _edge_combine(p, cnts):
    return pl.pallas_call(
        _edge_combine_body,
        grid=(EP // BRE,),
        in_specs=[
            pl.BlockSpec((2, BRE, F), lambda i: (0, i, 0)),
            pl.BlockSpec((2, BRE, 16), lambda i: (0, i, 0)),
        ],
        out_specs=pl.BlockSpec((BRE, F), lambda i: (i, 0)),
        out_shape=jax.ShapeDtypeStruct((EP, F), jnp.float32),
    )(p, cnts)


def _vert_combine_body(p_ref, h0_ref, w_ref, sc_ref, o_ref):
    y = p_ref[0] + p_ref[1]
    mu = jnp.mean(y, axis=1, keepdims=True)
    d = y - mu
    var = jnp.mean(d * d, axis=1, keepdims=True)
    xn = d * lax.rsqrt(var + EPS)
    xn = xn * sc_ref[2:3, :] + sc_ref[3:4, :]
    alpha = sc_ref[0:1, :]
    beta = sc_ref[1:2, :]
    xi = (1.0 - alpha) * xn + alpha * h0_ref[...]
    z = (1.0 - beta) * xi + beta * lax.dot_general(
        xi, w_ref[...], (((1,), (1,)), ((), ())),
        preferred_element_type=jnp.float32)
    o_ref[...] = z * jax.nn.sigmoid(z)


def _vert_combine(p, h0, w, sc):
    return pl.pallas_call(
        _vert_combine_body,
        grid=(VP // BR,),
        in_specs=[
            pl.BlockSpec((2, BR, F), lambda i: (0, i, 0)),
            pl.BlockSpec((BR, F), lambda i: (i, 0)),
            pl.BlockSpec((F, F), lambda i: (0, 0)),
            pl.BlockSpec((4, F), lambda i: (0, 0)),
        ],
        out_specs=pl.BlockSpec((BR, F), lambda i: (i, 0)),
        out_shape=jax.ShapeDtypeStruct((VP, F), jnp.float32),
    )(p, h0, w, sc)


def _out_body(h_ref, w_ref, b_ref, o_ref):
    z = lax.dot_general(h_ref[...], w_ref[...], (((1,), (1,)), ((), ())),
                        preferred_element_type=jnp.float32)
    z = z + b_ref[...]
    m = jnp.max(z, axis=1, keepdims=True)
    e = z - m
    lse = jnp.log(jnp.sum(jnp.exp(e), axis=1, keepdims=True))
    o_ref[...] = e - lse


def _out_linear(h, w, b):
    return pl.pallas_call(
        _out_body,
        grid=(VP // BR,),
        in_specs=[
            pl.BlockSpec((BR, F), lambda i: (i, 0)),
            pl.BlockSpec((NCLASS, F), lambda i: (0, 0)),
            pl.BlockSpec((1, NCLASS), lambda i: (0, 0)),
        ],
        out_specs=pl.BlockSpec((BR, NCLASS), lambda i: (i, 0)),
        out_shape=jax.ShapeDtypeStruct((VP, NCLASS), jnp.float32),
    )(h, w, b)


def kernel(x, V, E, W_in, b_in, W0, alpha0, beta0, g0, bln0,
           W1, alpha1, beta1, g1, bln1, W_out, b_out):
    xp = jnp.zeros((VP, F), jnp.float32).at[:N].set(x)
    zeros = jnp.zeros((VP, F), jnp.float32)
    zeros16 = jnp.zeros((EP, 16), jnp.float32)
    ones = jnp.ones((C, 16), jnp.float32)

    h0 = _lin_silu(xp, W_in, b_in.reshape(1, F))

    sc0 = jnp.stack([jnp.broadcast_to(alpha0, (F,)),
                     jnp.broadcast_to(beta0, (F,)), g0, bln0])
    sc1 = jnp.stack([jnp.broadcast_to(alpha1, (F,)),
                     jnp.broadcast_to(beta1, (F,)), g1, bln1])

    p_e, cnts = _seg_edge_cnt(h0, V, E, zeros, ones, zeros16)
    xe = _edge_combine(p_e, cnts)
    (p_v,) = _seg_vert(xe, E, V, zeros)
    h1 = _vert_combine(p_v, h0, W0, sc0)

    (p_e1,) = _seg_edge(h1, V, E, zeros)
    xe1 = _edge_combine(p_e1, cnts)
    (p_v1,) = _seg_vert(xe1, E, V, zeros)
    h2 = _vert_combine(p_v1, h0, W1, sc1)

    out = _out_linear(h2, W_out, b_out.reshape(1, NCLASS))
    return out[:N]


# speedup vs baseline: 8.9030x; 1.8936x over previous
import functools

import jax
import jax.numpy as jnp
from jax import lax
from jax.experimental import pallas as pl
from jax.experimental.pallas import tpu as pltpu, tpu_sc as plsc

N = 10000
NNZ = 320000
NE = 5000
F = 128
NCLASS = 16
VP = 10240
EP = 5120
DEGE = 0.5
EPS = 1e-5

NTILES = 32
PAIRS_PER_TILE = NNZ // NTILES
C = 80
NCHUNK = PAIRS_PER_TILE // C
NBUF = 4
NGROUP = (NCHUNK + 2 + NBUF - 1) // NBUF

_mesh = plsc.VectorSubcoreMesh(core_axis_name="c", subcore_axis_name="s")


def _make_seg_kernel(ndst, with_counts):
    rows_per_tile = ndst // 16
    crows = EP // 16
    out_type = [jax.ShapeDtypeStruct((2, ndst, F), jnp.float32)]
    scratch = [
        pltpu.VMEM_SHARED((ndst, F), jnp.float32),
    ]
    scratch += [pltpu.VMEM((C,), jnp.int32) for _ in range(NBUF)]
    scratch += [pltpu.VMEM((C,), jnp.int32) for _ in range(NBUF)]
    scratch += [pltpu.VMEM((C, F), jnp.float32) for _ in range(NBUF)]
    scratch += [pltpu.SemaphoreType.DMA for _ in range(3 * NBUF)]
    if with_counts:
        out_type.append(jax.ShapeDtypeStruct((2, EP, 16), jnp.float32))
        scratch += [
            pltpu.VMEM_SHARED((EP, 16), jnp.float32),
            pltpu.VMEM((C, 16), jnp.float32),
        ]
        scratch += [pltpu.SemaphoreType.DMA for _ in range(NBUF)]

    def body(table, src_idx, dst_idx, zeros, *rest):
        if with_counts:
            ones, zeros16, out, cnt_out = rest[:4]
            rest = rest[4:]
        else:
            out = rest[0]
            rest = rest[1:]
        accum = rest[0]
        sidx = rest[1:1 + NBUF]
        didx = rest[1 + NBUF:1 + 2 * NBUF]
        rows = rest[1 + 2 * NBUF:1 + 3 * NBUF]
        semi = rest[1 + 3 * NBUF:1 + 4 * NBUF]
        semg = rest[1 + 4 * NBUF:1 + 5 * NBUF]
        sems = rest[1 + 5 * NBUF:1 + 6 * NBUF]
        if with_counts:
            cnt_sh, ones_v = rest[1 + 6 * NBUF:3 + 6 * NBUF]
            semc = rest[3 + 6 * NBUF:3 + 7 * NBUF]
        core = lax.axis_index("c")
        sub = lax.axis_index("s")
        wid = core * 16 + sub
        r0 = sub * rows_per_tile
        pltpu.sync_copy(zeros.at[pl.ds(r0, rows_per_tile)],
                        accum.at[pl.ds(r0, rows_per_tile)])
        if with_counts:
            pltpu.sync_copy(zeros16.at[pl.ds(sub * crows, crows)],
                            cnt_sh.at[pl.ds(sub * crows, crows)])
            pltpu.sync_copy(ones, ones_v)
        plsc.subcore_barrier()

        base0 = wid * PAIRS_PER_TILE

        def stages(g, carry):
            for i in range(NBUF):
                j = g * NBUF + i
                s2 = (i - 2) % NBUF

                @pl.when(jnp.logical_and(j >= 2, j - 2 < NCHUNK))
                def _():
                    pltpu.make_async_copy(table.at[sidx[s2]], rows[s2],
                                          semg[s2]).wait()
                    pltpu.async_copy(rows[s2], accum.at[didx[s2]], sems[s2],
                                     add=True)
                    if with_counts:
                        pltpu.async_copy(ones_v, cnt_sh.at[didx[s2]], semc[s2],
                                         add=True)

                s1 = (i - 1) % NBUF

                @pl.when(jnp.logical_and(j >= 1, j - 1 < NCHUNK))
                def _():
                    pltpu.make_async_copy(src_idx.at[pl.ds(0, C)], sidx[s1],
                                          semi[s1]).wait()
                    pltpu.make_async_copy(dst_idx.at[pl.ds(0, C)], didx[s1],
                                          semi[s1]).wait()
                    pltpu.async_copy(table.at[sidx[s1]], rows[s1], semg[s1])

                @pl.when(j < NCHUNK)
                def _():
                    @pl.when(j >= NBUF)
                    def _():
                        pltpu.make_async_copy(rows[i], accum.at[didx[i]],
                                              sems[i]).wait()
                        if with_counts:
                            pltpu.make_async_copy(ones_v, cnt_sh.at[didx[i]],
                                                  semc[i]).wait()
                    base = base0 + j * C
                    pltpu.async_copy(src_idx.at[pl.ds(base, C)], sidx[i], semi[i])
                    pltpu.async_copy(dst_idx.at[pl.ds(base, C)], didx[i], semi[i])
            return carry

        lax.fori_loop(0, NGROUP, stages, 0)
        for k in range(NCHUNK - NBUF, NCHUNK):
            s = k % NBUF
            pltpu.make_async_copy(rows[s], accum.at[didx[s]], sems[s]).wait()
            if with_counts:
                pltpu.make_async_copy(ones_v, cnt_sh.at[didx[s]], semc[s]).wait()
        plsc.subcore_barrier()
        pltpu.sync_copy(accum.at[pl.ds(r0, rows_per_tile)],
                        out.at[core, pl.ds(r0, rows_per_tile)])
        if with_counts:
            pltpu.sync_copy(cnt_sh.at[pl.ds(sub * crows, crows)],
                            cnt_out.at[core, pl.ds(sub * crows, crows)])

    return pl.kernel(body, mesh=_mesh, out_type=out_type, scratch_types=scratch)


_seg_edge_cnt = _make_seg_kernel(EP, True)
_seg_edge = _make_seg_kernel(EP, False)
_seg_vert = _make_seg_kernel(VP, False)


BR = 2048
BRE = 1024


def _lin_silu_body(x_ref, w_ref, b_ref, o_ref):
    z = lax.dot_general(x_ref[...], w_ref[...], (((1,), (1,)), ((), ())),
                        preferred_element_type=jnp.float32)
    z = z + b_ref[...]
    o_ref[...] = z * jax.nn.sigmoid(z)


def _lin_silu(x, w, b):
    return pl.pallas_call(
        _lin_silu_body,
        grid=(VP // BR,),
        in_specs=[
            pl.BlockSpec((BR, F), lambda i: (i, 0)),
            pl.BlockSpec((F, F), lambda i: (0, 0)),
            pl.BlockSpec((1, F), lambda i: (0, 0)),
        ],
        out_specs=pl.BlockSpec((BR, F), lambda i: (i, 0)),
        out_shape=jax.ShapeDtypeStruct((VP, F), jnp.float32),
    )(x, w, b)


def _edge_combine_body(p_ref, c_ref, o_ref):
    s = p_ref[0] + p_ref[1]
    cnt = c_ref[0, :, 0:1] + c_ref[1, :, 0:1]
    o_ref[...] = s * (DEGE / jnp.maximum(cnt, 1.0))


def _edge_combine(p, cnts):
    return pl.pallas_call(
        _edge_combine_body,
        grid=(EP // BRE,),
        in_specs=[
            pl.BlockSpec((2, BRE, F), lambda i: (0, i, 0)),
            pl.BlockSpec((2, BRE, 16), lambda i: (0, i, 0)),
        ],
        out_specs=pl.BlockSpec((BRE, F), lambda i: (i, 0)),
        out_shape=jax.ShapeDtypeStruct((EP, F), jnp.float32),
    )(p, cnts)


def _vert_combine_body(p_ref, h0_ref, w_ref, sc_ref, o_ref):
    y = p_ref[0] + p_ref[1]
    mu = jnp.mean(y, axis=1, keepdims=True)
    d = y - mu
    var = jnp.mean(d * d, axis=1, keepdims=True)
    xn = d * lax.rsqrt(var + EPS)
    xn = xn * sc_ref[2:3, :] + sc_ref[3:4, :]
    alpha = sc_ref[0:1, :]
    beta = sc_ref[1:2, :]
    xi = (1.0 - alpha) * xn + alpha * h0_ref[...]
    z = (1.0 - beta) * xi + beta * lax.dot_general(
        xi, w_ref[...], (((1,), (1,)), ((), ())),
        preferred_element_type=jnp.float32)
    o_ref[...] = z * jax.nn.sigmoid(z)


def _vert_combine(p, h0, w, sc):
    return pl.pallas_call(
        _vert_combine_body,
        grid=(VP // BR,),
        in_specs=[
            pl.BlockSpec((2, BR, F), lambda i: (0, i, 0)),
            pl.BlockSpec((BR, F), lambda i: (i, 0)),
            pl.BlockSpec((F, F), lambda i: (0, 0)),
            pl.BlockSpec((4, F), lambda i: (0, 0)),
        ],
        out_specs=pl.BlockSpec((BR, F), lambda i: (i, 0)),
        out_shape=jax.ShapeDtypeStruct((VP, F), jnp.float32),
    )(p, h0, w, sc)


def _out_body(h_ref, w_ref, b_ref, o_ref):
    z = lax.dot_general(h_ref[...], w_ref[...], (((1,), (1,)), ((), ())),
                        preferred_element_type=jnp.float32)
    z = z + b_ref[...]
    m = jnp.max(z, axis=1, keepdims=True)
    e = z - m
    lse = jnp.log(jnp.sum(jnp.exp(e), axis=1, keepdims=True))
    o_ref[...] = e - lse


def _out_linear(h, w, b):
    return pl.pallas_call(
        _out_body,
        grid=(VP // BR,),
        in_specs=[
            pl.BlockSpec((BR, F), lambda i: (i, 0)),
            pl.BlockSpec((NCLASS, F), lambda i: (0, 0)),
            pl.BlockSpec((1, NCLASS), lambda i: (0, 0)),
        ],
        out_specs=pl.BlockSpec((BR, NCLASS), lambda i: (i, 0)),
        out_shape=jax.ShapeDtypeStruct((VP, NCLASS), jnp.float32),
    )(h, w, b)


def kernel(x, V, E, W_in, b_in, W0, alpha0, beta0, g0, bln0,
           W1, alpha1, beta1, g1, bln1, W_out, b_out):
    xp = jnp.zeros((VP, F), jnp.float32).at[:N].set(x)
    zeros = jnp.zeros((VP, F), jnp.float32)
    zeros16 = jnp.zeros((EP, 16), jnp.float32)
    ones = jnp.ones((C, 16), jnp.float32)

    h0 = _lin_silu(xp, W_in, b_in.reshape(1, F))

    sc0 = jnp.stack([jnp.broadcast_to(alpha0, (F,)),
                     jnp.broadcast_to(beta0, (F,)), g0, bln0])
    sc1 = jnp.stack([jnp.broadcast_to(alpha1, (F,)),
                     jnp.broadcast_to(beta1, (F,)), g1, bln1])

    p_e, cnts = _seg_edge_cnt(h0, V, E, zeros, ones, zeros16)
    xe = _edge_combine(p_e, cnts)
    (p_v,) = _seg_vert(xe, E, V, zeros)
    h1 = _vert_combine(p_v, h0, W0, sc0)

    (p_e1,) = _seg_edge(h1, V, E, zeros)
    xe1 = _edge_combine(p_e1, cnts)
    (p_v1,) = _seg_vert(xe1, E, V, zeros)
    h2 = _vert_combine(p_v1, h0, W1, sc1)

    out = _out_linear(h2, W_out, b_out.reshape(1, NCLASS))
    return out[:N]


# speedup vs baseline: 11.4326x; 1.2841x over previous
import functools

import jax
import jax.numpy as jnp
from jax import lax
from jax.experimental import pallas as pl
from jax.experimental.pallas import tpu as pltpu, tpu_sc as plsc

N = 10000
NNZ = 320000
NE = 5000
F = 128
NCLASS = 16
VP = 10240
EP = 5120
DEGE = 0.5
EPS = 1e-5

NTILES = 32
NBUF = 4
NNZP = 327680

_mesh = plsc.VectorSubcoreMesh(core_axis_name="c", subcore_axis_name="s")


def _make_seg_kernel(ndst, C, pairs_per_tile):
    rows_per_tile = ndst // 16
    nchunk = pairs_per_tile // C
    assert pairs_per_tile % C == 0 and nchunk % NBUF == 0 and C % 16 == 0
    assert NTILES * pairs_per_tile >= NNZ and NTILES * pairs_per_tile <= NNZP
    out_type = [jax.ShapeDtypeStruct((2, ndst, F), jnp.float32)]
    scratch = [
        pltpu.VMEM_SHARED((ndst, F), jnp.float32),
    ]
    scratch += [pltpu.VMEM((C,), jnp.int32) for _ in range(NBUF)]
    scratch += [pltpu.VMEM((C,), jnp.int32) for _ in range(NBUF)]
    scratch += [pltpu.VMEM((C, F), jnp.float32) for _ in range(NBUF)]
    scratch += [pltpu.SemaphoreType.DMA for _ in range(3 * NBUF)]

    def body(table, src_idx, dst_idx, zeros, out, *rest):
        accum = rest[0]
        sidx = rest[1:1 + NBUF]
        didx = rest[1 + NBUF:1 + 2 * NBUF]
        rows = rest[1 + 2 * NBUF:1 + 3 * NBUF]
        semi = rest[1 + 3 * NBUF:1 + 4 * NBUF]
        semg = rest[1 + 4 * NBUF:1 + 5 * NBUF]
        sems = rest[1 + 5 * NBUF:1 + 6 * NBUF]
        core = lax.axis_index("c")
        sub = lax.axis_index("s")
        wid = core * 16 + sub
        r0 = sub * rows_per_tile
        pltpu.sync_copy(zeros.at[pl.ds(r0, rows_per_tile)],
                        accum.at[pl.ds(r0, rows_per_tile)])
        plsc.subcore_barrier()

        base0 = wid * pairs_per_tile

        def issue_idx(s, k):
            base = base0 + k * C
            pltpu.async_copy(src_idx.at[pl.ds(base, C)], sidx[s], semi[s])
            pltpu.async_copy(dst_idx.at[pl.ds(base, C)], didx[s], semi[s])

        def issue_gather(s):
            pltpu.make_async_copy(src_idx.at[pl.ds(0, C)], sidx[s], semi[s]).wait()
            pltpu.make_async_copy(dst_idx.at[pl.ds(0, C)], didx[s], semi[s]).wait()
            pltpu.async_copy(table.at[sidx[s]], rows[s], semg[s])

        def issue_scatter(s):
            pltpu.make_async_copy(table.at[sidx[s]], rows[s], semg[s]).wait()
            pltpu.async_copy(rows[s], accum.at[didx[s]], sems[s], add=True)

        def wait_reuse(s):
            pltpu.make_async_copy(rows[s], accum.at[didx[s]], sems[s]).wait()

        issue_idx(0, 0)
        issue_idx(1, 1)
        issue_gather(0)
        issue_idx(2, 2)
        issue_gather(1)
        issue_scatter(0)
        issue_idx(3, 3)
        issue_gather(2)
        issue_scatter(1)

        def stages(g, carry):
            for i in range(NBUF):
                j = g * NBUF + i
                wait_reuse(i)
                issue_idx(i, j)
                issue_gather((i - 1) % NBUF)
                issue_scatter((i - 2) % NBUF)
            return carry

        lax.fori_loop(1, nchunk // NBUF, stages, 0)

        issue_gather((nchunk - 1) % NBUF)
        issue_scatter((nchunk - 2) % NBUF)
        issue_scatter((nchunk - 1) % NBUF)
        for k in range(nchunk - NBUF, nchunk):
            wait_reuse(k % NBUF)
        plsc.subcore_barrier()
        pltpu.sync_copy(accum.at[pl.ds(r0, rows_per_tile)],
                        out.at[core, pl.ds(r0, rows_per_tile)])

    return pl.kernel(body, mesh=_mesh, out_type=out_type, scratch_types=scratch)


def _make_cnt_kernel(C, pairs_per_tile):
    rows_per_tile = EP // 16
    nchunk = pairs_per_tile // C
    assert pairs_per_tile % C == 0 and nchunk % NBUF == 0 and C % 16 == 0
    out_type = [jax.ShapeDtypeStruct((2, EP, F), jnp.float32)]
    scratch = [pltpu.VMEM_SHARED((EP, F), jnp.float32)]
    scratch += [pltpu.VMEM((C,), jnp.int32) for _ in range(NBUF)]
    scratch += [pltpu.VMEM((C, F), jnp.float32)]
    scratch += [pltpu.SemaphoreType.DMA for _ in range(2 * NBUF)]

    def body(dst_idx, zeros, ones, out, *rest):
        accum = rest[0]
        didx = rest[1:1 + NBUF]
        ones_v = rest[1 + NBUF]
        semi = rest[2 + NBUF:2 + 2 * NBUF]
        sems = rest[2 + 2 * NBUF:2 + 3 * NBUF]
        core = lax.axis_index("c")
        sub = lax.axis_index("s")
        wid = core * 16 + sub
        r0 = sub * rows_per_tile
        pltpu.sync_copy(zeros.at[pl.ds(r0, rows_per_tile)],
                        accum.at[pl.ds(r0, rows_per_tile)])
        pltpu.sync_copy(ones, ones_v)
        plsc.subcore_barrier()

        base0 = wid * pairs_per_tile

        def issue_idx(s, k):
            pltpu.async_copy(dst_idx.at[pl.ds(base0 + k * C, C)], didx[s],
                             semi[s])

        def issue_scatter(s):
            pltpu.make_async_copy(dst_idx.at[pl.ds(0, C)], didx[s],
                                  semi[s]).wait()
            pltpu.async_copy(ones_v, accum.at[didx[s]], sems[s], add=True)

        def wait_reuse(s):
            pltpu.make_async_copy(ones_v, accum.at[didx[s]], sems[s]).wait()

        issue_idx(0, 0)
        issue_idx(1, 1)
        issue_scatter(0)
        issue_idx(2, 2)
        issue_scatter(1)
        issue_idx(3, 3)
        issue_scatter(2)

        def stages(g, carry):
            for i in range(NBUF):
                j = g * NBUF + i
                wait_reuse(i)
                issue_idx(i, j)
                issue_scatter((i - 1) % NBUF)
            return carry

        lax.fori_loop(1, nchunk // NBUF, stages, 0)
        issue_scatter((nchunk - 1) % NBUF)
        for k in range(nchunk - NBUF, nchunk):
            wait_reuse(k % NBUF)
        plsc.subcore_barrier()
        pltpu.sync_copy(accum.at[pl.ds(r0, rows_per_tile)],
                        out.at[core, pl.ds(r0, rows_per_tile)])

    return pl.kernel(body, mesh=_mesh, out_type=out_type, scratch_types=scratch)


CE = 80
CV = 80
_seg_edge = _make_seg_kernel(EP, CE, 10240)
_seg_vert = _make_seg_kernel(VP, CV, 10240)
_cnt_edge = _make_cnt_kernel(CE, 10240)


BR = 2048
BRE = 1024


def _lin_silu_body(x_ref, w_ref, b_ref, o_ref):
    z = lax.dot_general(x_ref[...].astype(jnp.bfloat16),
                        w_ref[...].astype(jnp.bfloat16),
                        (((1,), (1,)), ((), ())),
                        preferred_element_type=jnp.float32)
    z = z + b_ref[...]
    o_ref[...] = z * jax.nn.sigmoid(z)


def _lin_silu(x, w, b):
    return pl.pallas_call(
        _lin_silu_body,
        grid=(VP // BR,),
        in_specs=[
            pl.BlockSpec((BR, F), lambda i: (i, 0)),
            pl.BlockSpec((F, F), lambda i: (0, 0)),
            pl.BlockSpec((1, F), lambda i: (0, 0)),
        ],
        out_specs=pl.BlockSpec((BR, F), lambda i: (i, 0)),
        out_shape=jax.ShapeDtypeStruct((VP, F), jnp.float32),
    )(x, w, b)


def _edge_combine_body(p_ref, c_ref, o_ref):
    s = p_ref[0] + p_ref[1]
    cnt = c_ref[0, :, 0:1] + c_ref[1, :, 0:1]
    o_ref[...] = s * (DEGE / jnp.maximum(cnt, 1.0))


def _edge_combine(p, cnts):
    return pl.pallas_call(
        _edge_combine_body,
        grid=(EP // BRE,),
        in_specs=[
            pl.BlockSpec((2, BRE, F), lambda i: (0, i, 0)),
            pl.BlockSpec((2, BRE, F), lambda i: (0, i, 0)),
        ],
        out_specs=pl.BlockSpec((BRE, F), lambda i: (i, 0)),
        out_shape=jax.ShapeDtypeStruct((EP, F), jnp.float32),
    )(p, cnts)


def _vert_combine_body(p_ref, h0_ref, w_ref, sc_ref, o_ref):
    y = p_ref[0] + p_ref[1]
    mu = jnp.mean(y, axis=1, keepdims=True)
    d = y - mu
    var = jnp.mean(d * d, axis=1, keepdims=True)
    xn = d * lax.rsqrt(var + EPS)
    xn = xn * sc_ref[2:3, :] + sc_ref[3:4, :]
    alpha = sc_ref[0:1, :]
    beta = sc_ref[1:2, :]
    xi = (1.0 - alpha) * xn + alpha * h0_ref[...]
    z = (1.0 - beta) * xi + beta * lax.dot_general(
        xi.astype(jnp.bfloat16), w_ref[...].astype(jnp.bfloat16),
        (((1,), (1,)), ((), ())),
        preferred_element_type=jnp.float32)
    o_ref[...] = z * jax.nn.sigmoid(z)


def _vert_combine(p, h0, w, sc):
    return pl.pallas_call(
        _vert_combine_body,
        grid=(VP // BR,),
        in_specs=[
            pl.BlockSpec((2, BR, F), lambda i: (0, i, 0)),
            pl.BlockSpec((BR, F), lambda i: (i, 0)),
            pl.BlockSpec((F, F), lambda i: (0, 0)),
            pl.BlockSpec((4, F), lambda i: (0, 0)),
        ],
        out_specs=pl.BlockSpec((BR, F), lambda i: (i, 0)),
        out_shape=jax.ShapeDtypeStruct((VP, F), jnp.float32),
    )(p, h0, w, sc)


def _out_body(h_ref, w_ref, b_ref, o_ref):
    z = lax.dot_general(h_ref[...].astype(jnp.bfloat16),
                        w_ref[...].astype(jnp.bfloat16),
                        (((1,), (1,)), ((), ())),
                        preferred_element_type=jnp.float32)
    z = z + b_ref[...]
    m = jnp.max(z, axis=1, keepdims=True)
    e = z - m
    lse = jnp.log(jnp.sum(jnp.exp(e), axis=1, keepdims=True))
    o_ref[...] = e - lse


def _out_linear(h, w, b):
    return pl.pallas_call(
        _out_body,
        grid=(VP // BR,),
        in_specs=[
            pl.BlockSpec((BR, F), lambda i: (i, 0)),
            pl.BlockSpec((NCLASS, F), lambda i: (0, 0)),
            pl.BlockSpec((1, NCLASS), lambda i: (0, 0)),
        ],
        out_specs=pl.BlockSpec((BR, NCLASS), lambda i: (i, 0)),
        out_shape=jax.ShapeDtypeStruct((VP, NCLASS), jnp.float32),
    )(h, w, b)


def kernel(x, V, E, W_in, b_in, W0, alpha0, beta0, g0, bln0,
           W1, alpha1, beta1, g1, bln1, W_out, b_out):
    xp = jnp.zeros((VP, F), jnp.float32).at[:N].set(x)
    zeros = jnp.zeros((VP, F), jnp.float32)
    ones = jnp.ones((CE, F), jnp.float32)

    pad = jnp.arange(NNZP - NNZ, dtype=jnp.int32)
    Vp = jnp.concatenate([V, N + pad % (VP - N)])
    Ep = jnp.concatenate([E, NE + pad % (EP - NE)])

    h0 = _lin_silu(xp, W_in, b_in.reshape(1, F))

    sc0 = jnp.stack([jnp.broadcast_to(alpha0, (F,)),
                     jnp.broadcast_to(beta0, (F,)), g0, bln0])
    sc1 = jnp.stack([jnp.broadcast_to(alpha1, (F,)),
                     jnp.broadcast_to(beta1, (F,)), g1, bln1])

    (cnts,) = _cnt_edge(Ep, zeros, ones)
    (p_e,) = _seg_edge(h0, Vp, Ep, zeros)
    xe = _edge_combine(p_e, cnts)
    (p_v,) = _seg_vert(xe, Ep, Vp, zeros)
    h1 = _vert_combine(p_v, h0, W0, sc0)

    (p_e1,) = _seg_edge(h1, Vp, Ep, zeros)
    xe1 = _edge_combine(p_e1, cnts)
    (p_v1,) = _seg_vert(xe1, Ep, Vp, zeros)
    h2 = _vert_combine(p_v1, h0, W1, sc1)

    out = _out_linear(h2, W_out, b_out.reshape(1, NCLASS))
    return out[:N]


# speedup vs baseline: 11.6600x; 1.0199x over previous
import functools

import jax
import jax.numpy as jnp
from jax import lax
from jax.experimental import pallas as pl
from jax.experimental.pallas import tpu as pltpu, tpu_sc as plsc

N = 10000
NNZ = 320000
NE = 5000
F = 128
NCLASS = 16
VP = 10240
EP = 5120
DEGE = 0.5
EPS = 1e-5

NTILES = 32
NBUF = 4
NNZP = 327680

_mesh = plsc.VectorSubcoreMesh(core_axis_name="c", subcore_axis_name="s")


def _make_seg_kernel(ndst, C, pairs_per_tile, NBUF):
    rows_per_tile = ndst // 16
    nchunk = pairs_per_tile // C
    assert pairs_per_tile % C == 0 and nchunk % NBUF == 0 and C % 16 == 0
    assert NTILES * pairs_per_tile >= NNZ and NTILES * pairs_per_tile <= NNZP
    out_type = [jax.ShapeDtypeStruct((2, ndst, F), jnp.float32)]
    scratch = [
        pltpu.VMEM_SHARED((ndst, F), jnp.float32),
    ]
    scratch += [pltpu.VMEM((C,), jnp.int32) for _ in range(NBUF)]
    scratch += [pltpu.VMEM((C,), jnp.int32) for _ in range(NBUF)]
    scratch += [pltpu.VMEM((C, F), jnp.float32) for _ in range(NBUF)]
    scratch += [pltpu.SemaphoreType.DMA for _ in range(3 * NBUF)]

    def body(table, src_idx, dst_idx, zeros, out, *rest):
        accum = rest[0]
        sidx = rest[1:1 + NBUF]
        didx = rest[1 + NBUF:1 + 2 * NBUF]
        rows = rest[1 + 2 * NBUF:1 + 3 * NBUF]
        semi = rest[1 + 3 * NBUF:1 + 4 * NBUF]
        semg = rest[1 + 4 * NBUF:1 + 5 * NBUF]
        sems = rest[1 + 5 * NBUF:1 + 6 * NBUF]
        core = lax.axis_index("c")
        sub = lax.axis_index("s")
        wid = core * 16 + sub
        r0 = sub * rows_per_tile
        pltpu.sync_copy(zeros.at[pl.ds(r0, rows_per_tile)],
                        accum.at[pl.ds(r0, rows_per_tile)])
        plsc.subcore_barrier()

        base0 = wid * pairs_per_tile

        def issue_idx(s, k):
            base = base0 + k * C
            pltpu.async_copy(src_idx.at[pl.ds(base, C)], sidx[s], semi[s])
            pltpu.async_copy(dst_idx.at[pl.ds(base, C)], didx[s], semi[s])

        def issue_gather(s):
            pltpu.make_async_copy(src_idx.at[pl.ds(0, C)], sidx[s], semi[s]).wait()
            pltpu.make_async_copy(dst_idx.at[pl.ds(0, C)], didx[s], semi[s]).wait()
            pltpu.async_copy(table.at[sidx[s]], rows[s], semg[s])

        def issue_scatter(s):
            pltpu.make_async_copy(table.at[sidx[s]], rows[s], semg[s]).wait()
            pltpu.async_copy(rows[s], accum.at[didx[s]], sems[s], add=True)

        def wait_reuse(s):
            pltpu.make_async_copy(rows[s], accum.at[didx[s]], sems[s]).wait()

        for jj in range(NBUF):
            issue_idx(jj, jj)
            if jj >= 1:
                issue_gather(jj - 1)
            if jj >= 2:
                issue_scatter(jj - 2)

        def stages(g, carry):
            for i in range(NBUF):
                j = g * NBUF + i
                wait_reuse(i)
                issue_idx(i, j)
                issue_gather((i - 1) % NBUF)
                issue_scatter((i - 2) % NBUF)
            return carry

        lax.fori_loop(1, nchunk // NBUF, stages, 0)

        issue_gather((nchunk - 1) % NBUF)
        issue_scatter((nchunk - 2) % NBUF)
        issue_scatter((nchunk - 1) % NBUF)
        for k in range(nchunk - NBUF, nchunk):
            wait_reuse(k % NBUF)
        plsc.subcore_barrier()
        pltpu.sync_copy(accum.at[pl.ds(r0, rows_per_tile)],
                        out.at[core, pl.ds(r0, rows_per_tile)])

    return pl.kernel(body, mesh=_mesh, out_type=out_type, scratch_types=scratch)


def _make_cnt_kernel(C, pairs_per_tile, NBUF):
    rows_per_tile = EP // 16
    nchunk = pairs_per_tile // C
    assert pairs_per_tile % C == 0 and nchunk % NBUF == 0 and C % 16 == 0
    out_type = [jax.ShapeDtypeStruct((2, EP, F), jnp.float32)]
    scratch = [pltpu.VMEM_SHARED((EP, F), jnp.float32)]
    scratch += [pltpu.VMEM((C,), jnp.int32) for _ in range(NBUF)]
    scratch += [pltpu.VMEM((C, F), jnp.float32)]
    scratch += [pltpu.SemaphoreType.DMA for _ in range(2 * NBUF)]

    def body(dst_idx, zeros, ones, out, *rest):
        accum = rest[0]
        didx = rest[1:1 + NBUF]
        ones_v = rest[1 + NBUF]
        semi = rest[2 + NBUF:2 + 2 * NBUF]
        sems = rest[2 + 2 * NBUF:2 + 3 * NBUF]
        core = lax.axis_index("c")
        sub = lax.axis_index("s")
        wid = core * 16 + sub
        r0 = sub * rows_per_tile
        pltpu.sync_copy(zeros.at[pl.ds(r0, rows_per_tile)],
                        accum.at[pl.ds(r0, rows_per_tile)])
        pltpu.sync_copy(ones, ones_v)
        plsc.subcore_barrier()

        base0 = wid * pairs_per_tile

        def issue_idx(s, k):
            pltpu.async_copy(dst_idx.at[pl.ds(base0 + k * C, C)], didx[s],
                             semi[s])

        def issue_scatter(s):
            pltpu.make_async_copy(dst_idx.at[pl.ds(0, C)], didx[s],
                                  semi[s]).wait()
            pltpu.async_copy(ones_v, accum.at[didx[s]], sems[s], add=True)

        def wait_reuse(s):
            pltpu.make_async_copy(ones_v, accum.at[didx[s]], sems[s]).wait()

        for jj in range(NBUF):
            issue_idx(jj, jj)
            if jj >= 1:
                issue_scatter(jj - 1)

        def stages(g, carry):
            for i in range(NBUF):
                j = g * NBUF + i
                wait_reuse(i)
                issue_idx(i, j)
                issue_scatter((i - 1) % NBUF)
            return carry

        lax.fori_loop(1, nchunk // NBUF, stages, 0)
        issue_scatter((nchunk - 1) % NBUF)
        for k in range(nchunk - NBUF, nchunk):
            wait_reuse(k % NBUF)
        plsc.subcore_barrier()
        pltpu.sync_copy(accum.at[pl.ds(r0, rows_per_tile)],
                        out.at[core, pl.ds(r0, rows_per_tile)])

    return pl.kernel(body, mesh=_mesh, out_type=out_type, scratch_types=scratch)


CE = 80
CV = 80
_seg_edge = _make_seg_kernel(EP, CE, 10080, 6)
_seg_vert = _make_seg_kernel(VP, CV, 10240, 4)
_cnt_edge = _make_cnt_kernel(CE, 10080, 6)


BR = 2048
BRE = 1024


def _lin_silu_body(x_ref, w_ref, b_ref, o_ref):
    z = lax.dot_general(x_ref[...].astype(jnp.bfloat16),
                        w_ref[...].astype(jnp.bfloat16),
                        (((1,), (1,)), ((), ())),
                        preferred_element_type=jnp.float32)
    z = z + b_ref[...]
    o_ref[...] = z * jax.nn.sigmoid(z)


def _lin_silu(x, w, b):
    return pl.pallas_call(
        _lin_silu_body,
        grid=(VP // BR,),
        in_specs=[
            pl.BlockSpec((BR, F), lambda i: (i, 0)),
            pl.BlockSpec((F, F), lambda i: (0, 0)),
            pl.BlockSpec((1, F), lambda i: (0, 0)),
        ],
        out_specs=pl.BlockSpec((BR, F), lambda i: (i, 0)),
        out_shape=jax.ShapeDtypeStruct((VP, F), jnp.float32),
    )(x, w, b)


def _edge_combine_body(p_ref, c_ref, o_ref):
    s = p_ref[0] + p_ref[1]
    cnt = c_ref[0, :, 0:1] + c_ref[1, :, 0:1]
    o_ref[...] = s * (DEGE / jnp.maximum(cnt, 1.0))


def _edge_combine(p, cnts):
    return pl.pallas_call(
        _edge_combine_body,
        grid=(EP // BRE,),
        in_specs=[
            pl.BlockSpec((2, BRE, F), lambda i: (0, i, 0)),
            pl.BlockSpec((2, BRE, F), lambda i: (0, i, 0)),
        ],
        out_specs=pl.BlockSpec((BRE, F), lambda i: (i, 0)),
        out_shape=jax.ShapeDtypeStruct((EP, F), jnp.float32),
    )(p, cnts)


def _vert_combine_body(p_ref, h0_ref, w_ref, sc_ref, o_ref):
    y = p_ref[0] + p_ref[1]
    mu = jnp.mean(y, axis=1, keepdims=True)
    d = y - mu
    var = jnp.mean(d * d, axis=1, keepdims=True)
    xn = d * lax.rsqrt(var + EPS)
    xn = xn * sc_ref[2:3, :] + sc_ref[3:4, :]
    alpha = sc_ref[0:1, :]
    beta = sc_ref[1:2, :]
    xi = (1.0 - alpha) * xn + alpha * h0_ref[...]
    z = (1.0 - beta) * xi + beta * lax.dot_general(
        xi.astype(jnp.bfloat16), w_ref[...].astype(jnp.bfloat16),
        (((1,), (1,)), ((), ())),
        preferred_element_type=jnp.float32)
    o_ref[...] = z * jax.nn.sigmoid(z)


def _vert_combine(p, h0, w, sc):
    return pl.pallas_call(
        _vert_combine_body,
        grid=(VP // BR,),
        in_specs=[
            pl.BlockSpec((2, BR, F), lambda i: (0, i, 0)),
            pl.BlockSpec((BR, F), lambda i: (i, 0)),
            pl.BlockSpec((F, F), lambda i: (0, 0)),
            pl.BlockSpec((4, F), lambda i: (0, 0)),
        ],
        out_specs=pl.BlockSpec((BR, F), lambda i: (i, 0)),
        out_shape=jax.ShapeDtypeStruct((VP, F), jnp.float32),
    )(p, h0, w, sc)


def _vert_out_body(p_ref, h0_ref, w_ref, sc_ref, wo_ref, bo_ref, o_ref):
    y = p_ref[0] + p_ref[1]
    mu = jnp.mean(y, axis=1, keepdims=True)
    d = y - mu
    var = jnp.mean(d * d, axis=1, keepdims=True)
    xn = d * lax.rsqrt(var + EPS)
    xn = xn * sc_ref[2:3, :] + sc_ref[3:4, :]
    alpha = sc_ref[0:1, :]
    beta = sc_ref[1:2, :]
    xi = (1.0 - alpha) * xn + alpha * h0_ref[...]
    z = (1.0 - beta) * xi + beta * lax.dot_general(
        xi.astype(jnp.bfloat16), w_ref[...].astype(jnp.bfloat16),
        (((1,), (1,)), ((), ())),
        preferred_element_type=jnp.float32)
    h = z * jax.nn.sigmoid(z)
    zz = lax.dot_general(h.astype(jnp.bfloat16), wo_ref[...].astype(jnp.bfloat16),
                         (((1,), (1,)), ((), ())),
                         preferred_element_type=jnp.float32)
    zz = zz + bo_ref[...]
    m = jnp.max(zz, axis=1, keepdims=True)
    e = zz - m
    lse = jnp.log(jnp.sum(jnp.exp(e), axis=1, keepdims=True))
    o_ref[...] = e - lse


def _vert_out(p, h0, w, sc, wo, bo):
    return pl.pallas_call(
        _vert_out_body,
        grid=(VP // BR,),
        in_specs=[
            pl.BlockSpec((2, BR, F), lambda i: (0, i, 0)),
            pl.BlockSpec((BR, F), lambda i: (i, 0)),
            pl.BlockSpec((F, F), lambda i: (0, 0)),
            pl.BlockSpec((4, F), lambda i: (0, 0)),
            pl.BlockSpec((NCLASS, F), lambda i: (0, 0)),
            pl.BlockSpec((1, NCLASS), lambda i: (0, 0)),
        ],
        out_specs=pl.BlockSpec((BR, NCLASS), lambda i: (i, 0)),
        out_shape=jax.ShapeDtypeStruct((VP, NCLASS), jnp.float32),
    )(p, h0, w, sc, wo, bo)


def _out_body(h_ref, w_ref, b_ref, o_ref):
    z = lax.dot_general(h_ref[...].astype(jnp.bfloat16),
                        w_ref[...].astype(jnp.bfloat16),
                        (((1,), (1,)), ((), ())),
                        preferred_element_type=jnp.float32)
    z = z + b_ref[...]
    m = jnp.max(z, axis=1, keepdims=True)
    e = z - m
    lse = jnp.log(jnp.sum(jnp.exp(e), axis=1, keepdims=True))
    o_ref[...] = e - lse


def _out_linear(h, w, b):
    return pl.pallas_call(
        _out_body,
        grid=(VP // BR,),
        in_specs=[
            pl.BlockSpec((BR, F), lambda i: (i, 0)),
            pl.BlockSpec((NCLASS, F), lambda i: (0, 0)),
            pl.BlockSpec((1, NCLASS), lambda i: (0, 0)),
        ],
        out_specs=pl.BlockSpec((BR, NCLASS), lambda i: (i, 0)),
        out_shape=jax.ShapeDtypeStruct((VP, NCLASS), jnp.float32),
    )(h, w, b)


def kernel(x, V, E, W_in, b_in, W0, alpha0, beta0, g0, bln0,
           W1, alpha1, beta1, g1, bln1, W_out, b_out):
    xp = jnp.zeros((VP, F), jnp.float32).at[:N].set(x)
    zeros = jnp.zeros((VP, F), jnp.float32)
    ones = jnp.ones((CE, F), jnp.float32)

    pad = jnp.arange(NNZP - NNZ, dtype=jnp.int32)
    Vp = jnp.concatenate([V, N + pad % (VP - N)])
    Ep = jnp.concatenate([E, NE + pad % (EP - NE)])

    h0 = _lin_silu(xp, W_in, b_in.reshape(1, F))

    sc0 = jnp.stack([jnp.broadcast_to(alpha0, (F,)),
                     jnp.broadcast_to(beta0, (F,)), g0, bln0])
    sc1 = jnp.stack([jnp.broadcast_to(alpha1, (F,)),
                     jnp.broadcast_to(beta1, (F,)), g1, bln1])

    (cnts,) = _cnt_edge(Ep, zeros, ones)
    (p_e,) = _seg_edge(h0, Vp, Ep, zeros)
    xe = _edge_combine(p_e, cnts)
    (p_v,) = _seg_vert(xe, Ep, Vp, zeros)
    h1 = _vert_combine(p_v, h0, W0, sc0)

    (p_e1,) = _seg_edge(h1, Vp, Ep, zeros)
    xe1 = _edge_combine(p_e1, cnts)
    (p_v1,) = _seg_vert(xe1, Ep, Vp, zeros)
    out = _vert_out(p_v1, h0, W1, sc1, W_out, b_out.reshape(1, NCLASS))
    return out[:N]
